# baseline (speedup 1.0000x reference)
import functools

import jax
import jax.numpy as jnp
from jax import lax
from jax.experimental import pallas as pl
from jax.experimental.pallas import tpu as pltpu
from jax.experimental.pallas import tpu_sc as plsc

N = 10000
E = 320000
D = 128
B = 4096
NC, NS = 2, 16
NW = NC * NS
EC = 128
CW = 80
CW0 = 160
CW1 = 2 * CW - CW0
IG = 16
EP = NW * CW * EC
ACC_N = NS * 640
RPT = ACC_N // NS
TAIL = N - (NS - 1) * RPT
SINK = N

_mesh = plsc.VectorSubcoreMesh(
    core_axis_name="c", subcore_axis_name="s", num_cores=NC, num_subcores=NS)


def _hop_body(with_deg, h_hbm, src_hbm, dst_hbm, part_hbm, deg_hbm,
              src_idx, dst_idx, rows0, rows1, ones, degv, iov, acc, dega,
              sem0, sem1):
    c = lax.axis_index("c")
    s = lax.axis_index("s")
    w = s * NC + c

    def _zrows(i, carry):
        rows0[i // (D // 16), pl.ds((i % (D // 16)) * 16, 16)] = (
            jnp.zeros((16,), jnp.float32))
        return carry
    lax.fori_loop(0, EC * (D // 16), _zrows, 0)

    def _zones(i, carry):
        ones[pl.ds(i * 16, 16)] = jnp.ones((16,), jnp.float32)
        degv[pl.ds(i * 16, 16)] = jnp.zeros((16,), jnp.float32)
        return carry
    lax.fori_loop(0, EC // 16, _zones, 0)

    def _ziov(i, carry):
        iov[pl.ds(i * 16, 16)] = lax.iota(jnp.int32, 16) + (s * RPT + i * 16)
        return carry
    lax.fori_loop(0, EC // 16, _ziov, 0)

    def _zacc(i, carry):
        pltpu.sync_copy(rows0, acc.at[pl.ds(s * RPT + i * EC, EC)])
        if with_deg:
            pltpu.sync_copy(rows0.at[0], dega.at[pl.ds(s * RPT + i * EC, EC)])
        return carry
    lax.fori_loop(0, RPT // EC, _zacc, 0)

    plsc.subcore_barrier()

    base = jnp.where(c == 0, s * CW0, NS * CW0 + s * CW1)
    ng = jnp.where(c == 0, CW0 // IG, CW1 // IG)

    def _group(g, carry):
        pltpu.sync_copy(src_hbm.at[pl.ds(base + g * IG, IG)], src_idx)
        pltpu.sync_copy(dst_hbm.at[pl.ds(base + g * IG, IG)], dst_idx)
        pltpu.async_copy(h_hbm.at[src_idx.at[0]], rows0, sem0).wait()

        def _edge_pair(k, carry2):
            j0 = 2 * k
            d1 = pltpu.async_copy(h_hbm.at[src_idx.at[j0 + 1]], rows1, sem1)
            if with_deg:
                pltpu.sync_copy(ones, dega.at[dst_idx.at[j0]], add=True)
            pltpu.sync_copy(rows0, acc.at[dst_idx.at[j0]], add=True)
            d1.wait()

            @pl.when(k < IG // 2 - 1)
            def _():
                pltpu.async_copy(h_hbm.at[src_idx.at[j0 + 2]], rows0, sem0)
            if with_deg:
                pltpu.sync_copy(ones, dega.at[dst_idx.at[j0 + 1]], add=True)
            pltpu.sync_copy(rows1, acc.at[dst_idx.at[j0 + 1]], add=True)

            @pl.when(k < IG // 2 - 1)
            def _():
                pltpu.make_async_copy(
                    h_hbm.at[src_idx.at[0]], rows0, sem0).wait()
            return carry2
        lax.fori_loop(0, IG // 2, _edge_pair, 0)
        return carry
    lax.fori_loop(0, ng, _group, 0)

    plsc.subcore_barrier()

    @pl.when(s < NS - 1)
    def _():
        pltpu.sync_copy(acc.at[pl.ds(s * RPT, RPT)],
                        part_hbm.at[c].at[pl.ds(s * RPT, RPT)])
        if with_deg:
            pltpu.sync_copy(dega.at[pl.ds(s * RPT, RPT)], degv)
            pltpu.sync_copy(degv, deg_hbm.at[pl.ds(c * N + s * RPT, RPT)])

    @pl.when(s == NS - 1)
    def _():
        pltpu.sync_copy(acc.at[pl.ds((NS - 1) * RPT, TAIL)],
                        part_hbm.at[c].at[pl.ds((NS - 1) * RPT, TAIL)])
        if with_deg:
            pltpu.sync_copy(dega.at[pl.ds((NS - 1) * RPT, TAIL)],
                            degv.at[pl.ds(0, TAIL)])
            pltpu.sync_copy(degv.at[pl.ds(0, TAIL)],
                            deg_hbm.at[pl.ds(c * N + (NS - 1) * RPT, TAIL)])


def _make_hop(with_deg):
    return pl.kernel(
        functools.partial(_hop_body, with_deg),
        out_type=(jax.ShapeDtypeStruct((NC, N, D), jnp.float32),
                  jax.ShapeDtypeStruct((NC * N,), jnp.float32)),
        mesh=_mesh,
        scratch_types=[
            pltpu.VMEM((IG, EC), jnp.int32),
            pltpu.VMEM((IG, EC), jnp.int32),
            pltpu.VMEM((EC, D), jnp.float32),
            pltpu.VMEM((EC, D), jnp.float32),
            pltpu.VMEM((EC,), jnp.float32),
            pltpu.VMEM((RPT,), jnp.float32),
            pltpu.VMEM((EC,), jnp.int32),
            pltpu.VMEM_SHARED((ACC_N, D), jnp.float32),
            pltpu.VMEM_SHARED((ACC_N,), jnp.float32),
            pltpu.SemaphoreType.DMA,
            pltpu.SemaphoreType.DMA,
        ],
    )


_hop_deg = _make_hop(True)
_hop_nodeg = _make_hop(False)


def _combine_body(p_ref, d_ref, o_ref):
    d = d_ref[0, :] + d_ref[1, :]
    r = 1.0 / jnp.maximum(d, 1.0)
    o_ref[...] = (p_ref[0] + p_ref[1]) * r[:, None]


_combine = pl.pallas_call(
    _combine_body,
    out_shape=jax.ShapeDtypeStruct((N, D), jnp.float32),
)


def _query_body(h2_hbm, emb_hbm, ids_hbm, q0_hbm, q1_hbm, idv, r0, r1, sem):
    c = lax.axis_index("c")
    s = lax.axis_index("s")
    w = s * NC + c
    pltpu.sync_copy(ids_hbm.at[w], idv)
    pltpu.async_copy(h2_hbm.at[idv], r0, sem).wait()
    pltpu.sync_copy(r0, q0_hbm.at[pl.ds(w * (B // NW), B // NW)])
    pltpu.async_copy(emb_hbm.at[idv], r1, sem).wait()
    pltpu.sync_copy(r1, q1_hbm.at[pl.ds(w * (B // NW), B // NW)])


_query = pl.kernel(
    _query_body,
    out_type=(jax.ShapeDtypeStruct((B, D), jnp.float32),
              jax.ShapeDtypeStruct((B, D), jnp.float32)),
    mesh=_mesh,
    scratch_types=[
        pltpu.VMEM((B // NW,), jnp.int32),
        pltpu.VMEM((B // NW, D), jnp.float32),
        pltpu.VMEM((B // NW, D), jnp.float32),
        pltpu.SemaphoreType.DMA,
    ],
)


def _final_body(q0_ref, q1_ref, w_ref, b_ref, o_ref):
    f0 = jnp.dot(q0_ref[...], w_ref[...], preferred_element_type=jnp.float32)
    f1 = jnp.dot(q1_ref[...], b_ref[...], preferred_element_type=jnp.float32)
    o_ref[...] = jnp.maximum(f0 + f1, 0.0)


_final = pl.pallas_call(
    _final_body,
    out_shape=jax.ShapeDtypeStruct((B, D), jnp.float32),
)


def kernel(node_ids, edge_index, embed_table, weight, bias):
    src = edge_index[0]
    dst = edge_index[1]
    pad = EP - E
    srcp = jnp.concatenate(
        [src, jnp.zeros((pad,), jnp.int32)]).reshape(NW * CW, EC)
    dstp = jnp.concatenate(
        [dst, jnp.full((pad,), SINK, jnp.int32)]).reshape(NW * CW, EC)

    part1, deg1 = _hop_deg(embed_table, srcp, dstp)
    degs = deg1.reshape(NC, N)
    h1 = _combine(part1, degs)
    part2, _ = _hop_nodeg(h1, srcp, dstp)
    h2 = _combine(part2, degs)

    ids = node_ids.reshape(NW, B // NW)
    q0, q1 = _query(h2, embed_table, ids)
    return _final(q0, q1, weight, bias.reshape(D, 1))


# speedup vs baseline: 1.4581x; 1.4581x over previous
import functools

import jax
import jax.numpy as jnp
from jax import lax
from jax.experimental import pallas as pl
from jax.experimental.pallas import tpu as pltpu
from jax.experimental.pallas import tpu_sc as plsc

N = 10000
E = 320000
D = 128
B = 4096
NC, NS = 2, 16
NW = NC * NS
EC = 128
CW = 80
CW0 = 144
CW1 = 2 * CW - CW0
IG = 16
EP = NW * CW * EC
ACC_N = NS * 640
RPT = ACC_N // NS
TAIL = N - (NS - 1) * RPT
SINK = N

_mesh = plsc.VectorSubcoreMesh(
    core_axis_name="c", subcore_axis_name="s", num_cores=NC, num_subcores=NS)


def _hop_body(with_deg, h_hbm, src_hbm, dst_hbm, part_hbm, deg_hbm,
              src_idx, dst_idx, rows0, rows1, ones, degv, iov, acc, dega,
              sem0, sem1):
    c = lax.axis_index("c")
    s = lax.axis_index("s")
    w = s * NC + c

    def _zrows(i, carry):
        rows0[i // (D // 16), pl.ds((i % (D // 16)) * 16, 16)] = (
            jnp.zeros((16,), jnp.float32))
        return carry
    lax.fori_loop(0, EC * (D // 16), _zrows, 0)

    def _zones(i, carry):
        ones[pl.ds(i * 16, 16)] = jnp.ones((16,), jnp.float32)
        degv[pl.ds(i * 16, 16)] = jnp.zeros((16,), jnp.float32)
        return carry
    lax.fori_loop(0, EC // 16, _zones, 0)

    def _ziov(i, carry):
        iov[pl.ds(i * 16, 16)] = lax.iota(jnp.int32, 16) + (s * RPT + i * 16)
        return carry
    lax.fori_loop(0, EC // 16, _ziov, 0)

    def _zacc(i, carry):
        pltpu.sync_copy(rows0, acc.at[pl.ds(s * RPT + i * EC, EC)])
        if with_deg:
            pltpu.sync_copy(rows0.at[0], dega.at[pl.ds(s * RPT + i * EC, EC)])
        return carry
    lax.fori_loop(0, RPT // EC, _zacc, 0)

    plsc.subcore_barrier()

    base = jnp.where(c == 0, s * CW0, NS * CW0 + s * CW1)
    ng = jnp.where(c == 0, CW0 // IG, CW1 // IG)

    def _group(g, carry):
        pltpu.sync_copy(src_hbm.at[pl.ds(base + g * IG, IG)], src_idx)
        pltpu.sync_copy(dst_hbm.at[pl.ds(base + g * IG, IG)], dst_idx)
        pltpu.async_copy(h_hbm.at[src_idx.at[0]], rows0, sem0).wait()

        def _edge_pair(k, carry2):
            j0 = 2 * k
            d1 = pltpu.async_copy(h_hbm.at[src_idx.at[j0 + 1]], rows1, sem1)
            if with_deg:
                pltpu.sync_copy(ones, dega.at[dst_idx.at[j0]], add=True)
            pltpu.sync_copy(rows0, acc.at[dst_idx.at[j0]], add=True)
            d1.wait()

            @pl.when(k < IG // 2 - 1)
            def _():
                pltpu.async_copy(h_hbm.at[src_idx.at[j0 + 2]], rows0, sem0)
            if with_deg:
                pltpu.sync_copy(ones, dega.at[dst_idx.at[j0 + 1]], add=True)
            pltpu.sync_copy(rows1, acc.at[dst_idx.at[j0 + 1]], add=True)

            @pl.when(k < IG // 2 - 1)
            def _():
                pltpu.make_async_copy(
                    h_hbm.at[src_idx.at[0]], rows0, sem0).wait()
            return carry2
        lax.fori_loop(0, IG // 2, _edge_pair, 0)
        return carry
    lax.fori_loop(0, ng, _group, 0)

    plsc.subcore_barrier()

    @pl.when(s < NS - 1)
    def _():
        pltpu.sync_copy(acc.at[pl.ds(s * RPT, RPT)],
                        part_hbm.at[c].at[pl.ds(s * RPT, RPT)])
        if with_deg:
            pltpu.sync_copy(dega.at[pl.ds(s * RPT, RPT)], degv)
            pltpu.sync_copy(degv, deg_hbm.at[pl.ds(c * N + s * RPT, RPT)])

    @pl.when(s == NS - 1)
    def _():
        pltpu.sync_copy(acc.at[pl.ds((NS - 1) * RPT, TAIL)],
                        part_hbm.at[c].at[pl.ds((NS - 1) * RPT, TAIL)])
        if with_deg:
            pltpu.sync_copy(dega.at[pl.ds((NS - 1) * RPT, TAIL)],
                            degv.at[pl.ds(0, TAIL)])
            pltpu.sync_copy(degv.at[pl.ds(0, TAIL)],
                            deg_hbm.at[pl.ds(c * N + (NS - 1) * RPT, TAIL)])


def _make_hop(with_deg):
    return pl.kernel(
        functools.partial(_hop_body, with_deg),
        out_type=(jax.ShapeDtypeStruct((NC, N, D), jnp.float32),
                  jax.ShapeDtypeStruct((NC * N,), jnp.float32)),
        mesh=_mesh,
        scratch_types=[
            pltpu.VMEM((IG, EC), jnp.int32),
            pltpu.VMEM((IG, EC), jnp.int32),
            pltpu.VMEM((EC, D), jnp.float32),
            pltpu.VMEM((EC, D), jnp.float32),
            pltpu.VMEM((EC,), jnp.float32),
            pltpu.VMEM((RPT,), jnp.float32),
            pltpu.VMEM((EC,), jnp.int32),
            pltpu.VMEM_SHARED((ACC_N, D), jnp.float32),
            pltpu.VMEM_SHARED((ACC_N,), jnp.float32),
            pltpu.SemaphoreType.DMA,
            pltpu.SemaphoreType.DMA,
        ],
    )


_hop_deg = _make_hop(True)
_hop_nodeg = _make_hop(False)


def _combine_body(p_ref, d_ref, o_ref):
    d = d_ref[0, :] + d_ref[1, :]
    r = 1.0 / jnp.maximum(d, 1.0)
    o_ref[...] = (p_ref[0] + p_ref[1]) * r[:, None]


_combine = pl.pallas_call(
    _combine_body,
    out_shape=jax.ShapeDtypeStruct((N, D), jnp.float32),
)


def _query_body(h2_hbm, emb_hbm, ids_hbm, q0_hbm, q1_hbm, idv, r0, r1, sem):
    c = lax.axis_index("c")
    s = lax.axis_index("s")
    w = s * NC + c
    pltpu.sync_copy(ids_hbm.at[w], idv)
    pltpu.async_copy(h2_hbm.at[idv], r0, sem).wait()
    pltpu.sync_copy(r0, q0_hbm.at[pl.ds(w * (B // NW), B // NW)])
    pltpu.async_copy(emb_hbm.at[idv], r1, sem).wait()
    pltpu.sync_copy(r1, q1_hbm.at[pl.ds(w * (B // NW), B // NW)])


_query = pl.kernel(
    _query_body,
    out_type=(jax.ShapeDtypeStruct((B, D), jnp.float32),
              jax.ShapeDtypeStruct((B, D), jnp.float32)),
    mesh=_mesh,
    scratch_types=[
        pltpu.VMEM((B // NW,), jnp.int32),
        pltpu.VMEM((B // NW, D), jnp.float32),
        pltpu.VMEM((B // NW, D), jnp.float32),
        pltpu.SemaphoreType.DMA,
    ],
)


def _final_body(q0_ref, q1_ref, w_ref, b_ref, o_ref):
    f0 = jnp.dot(q0_ref[...], w_ref[...], preferred_element_type=jnp.float32)
    f1 = jnp.dot(q1_ref[...], b_ref[...], preferred_element_type=jnp.float32)
    o_ref[...] = jnp.maximum(f0 + f1, 0.0)


_final = pl.pallas_call(
    _final_body,
    out_shape=jax.ShapeDtypeStruct((B, D), jnp.float32),
)


def kernel(node_ids, edge_index, embed_table, weight, bias):
    src = edge_index[0]
    dst = edge_index[1]
    pad = EP - E
    srcp = jnp.concatenate(
        [src, jnp.zeros((pad,), jnp.int32)]).reshape(NW * CW, EC)
    dstp = jnp.concatenate(
        [dst, jnp.full((pad,), SINK, jnp.int32)]).reshape(NW * CW, EC)

    part1, deg1 = _hop_deg(embed_table, srcp, dstp)
    degs = deg1.reshape(NC, N)
    h1 = _combine(part1, degs)
    part2, _ = _hop_nodeg(h1, srcp, dstp)
    h2 = _combine(part2, degs)

    ids = node_ids.reshape(NW, B // NW)
    q0, q1 = _query(h2, embed_table, ids)
    return _final(q0, q1, weight, bias.reshape(D, 1))
